# Initial kernel scaffold; baseline (speedup 1.0000x reference)
import jax
import jax.numpy as jnp
from jax.experimental import pallas as pl


def kernel(x, edge_index, W1, b1, W2, b2, Wc, bc):
    raise NotImplementedError("write your pallas kernel here")


# speedup vs baseline: 10.3586x; 10.3586x over previous
import functools

import jax
import jax.numpy as jnp
from jax import lax
from jax.experimental import pallas as pl
from jax.experimental.pallas import tpu as pltpu
from jax.experimental.pallas import tpu_sc as plsc

N = 10000
E = 320000
D = 128
H = 16
C = 16

NC, NS = 2, 16
NW = NC * NS
TPW = E // NW
JW = 80
J = TPW // JW
NPAD = 10240
NPT = NPAD // NS

_mesh = plsc.VectorSubcoreMesh(core_axis_name="c", subcore_axis_name="s")
_sc_params = pltpu.CompilerParams(use_tc_tiling_on_sc=False)


def _acc_zero(s, zeros_hbm, obuf, acc):
    pltpu.sync_copy(zeros_hbm.at[pl.ds(s * NPT, NPT)], obuf)
    pltpu.sync_copy(obuf, acc.at[pl.ds(s * NPT, NPT)])
    plsc.subcore_barrier()


def _acc_dump(c, s, obuf, acc, out_hbm):
    plsc.subcore_barrier()
    pltpu.sync_copy(acc.at[pl.ds(s * NPT, NPT)], obuf)
    pltpu.sync_copy(obuf, out_hbm.at[c, pl.ds(s * NPT, NPT)])


@functools.partial(
    pl.kernel, mesh=_mesh, compiler_params=_sc_params,
    out_type=jax.ShapeDtypeStruct((NC, NPAD, H), jnp.float32),
    scratch_types=[
        pltpu.VMEM((J, JW), jnp.int32),
        pltpu.VMEM((NPT, H), jnp.float32),
        pltpu.VMEM((JW, H), jnp.float32),
        pltpu.VMEM_SHARED((NPAD, H), jnp.float32),
    ],
)
def _sc_deg(col_hbm, zeros_hbm, ones_hbm, out_hbm, idx_c, obuf, ones_v, acc):
    c, s = lax.axis_index("c"), lax.axis_index("s")
    wid = s * NC + c
    _acc_zero(s, zeros_hbm, obuf, acc)
    pltpu.sync_copy(ones_hbm, ones_v)
    pltpu.sync_copy(col_hbm.at[wid], idx_c)

    def body(j, carry):
        pltpu.sync_copy(ones_v, acc.at[idx_c.at[j]], add=True)
        return carry

    lax.fori_loop(0, J, body, 0)
    _acc_dump(c, s, obuf, acc, out_hbm)


@functools.partial(
    pl.kernel, mesh=_mesh, compiler_params=_sc_params,
    out_type=jax.ShapeDtypeStruct((NC, NPAD, H), jnp.float32),
    scratch_types=[
        pltpu.VMEM((J, JW), jnp.int32),
        pltpu.VMEM((J, JW), jnp.int32),
        pltpu.VMEM((NPT, H), jnp.float32),
        pltpu.VMEM((JW, H), jnp.float32),
        pltpu.VMEM_SHARED((NPAD, H), jnp.float32),
        pltpu.SemaphoreType.DMA,
    ],
)
def _sc_hop(zs_hbm, row_hbm, col_hbm, zeros_hbm, out_hbm,
            idx_r, idx_c, obuf, rows, acc, sem):
    c, s = lax.axis_index("c"), lax.axis_index("s")
    wid = s * NC + c
    _acc_zero(s, zeros_hbm, obuf, acc)
    pltpu.sync_copy(row_hbm.at[wid], idx_r)
    pltpu.sync_copy(col_hbm.at[wid], idx_c)

    def body(j, carry):
        pltpu.async_copy(zs_hbm.at[idx_r.at[j]], rows, sem).wait()
        pltpu.sync_copy(rows, acc.at[idx_c.at[j]], add=True)
        return carry

    lax.fori_loop(0, J, body, 0)
    _acc_dump(c, s, obuf, acc, out_hbm)


@functools.partial(
    pl.kernel, mesh=_mesh, compiler_params=_sc_params,
    out_type=(jax.ShapeDtypeStruct((E, H), jnp.float32),
              jax.ShapeDtypeStruct((E, H), jnp.float32)),
    scratch_types=[
        pltpu.VMEM((J, JW), jnp.int32),
        pltpu.VMEM((J, JW), jnp.int32),
        pltpu.VMEM((JW, H), jnp.float32),
        pltpu.VMEM((JW, H), jnp.float32),
        pltpu.SemaphoreType.DMA,
        pltpu.SemaphoreType.DMA,
    ],
)
def _sc_edge(a_hbm, b_hbm, row_hbm, col_hbm, ga_hbm, gb_hbm,
             idx_r, idx_c, ra, rb, sem_a, sem_b):
    c, s = lax.axis_index("c"), lax.axis_index("s")
    wid = s * NC + c
    pltpu.sync_copy(row_hbm.at[wid], idx_r)
    pltpu.sync_copy(col_hbm.at[wid], idx_c)
    base = wid * TPW

    def body(j, carry):
        da = pltpu.async_copy(a_hbm.at[idx_r.at[j]], ra, sem_a)
        db = pltpu.async_copy(b_hbm.at[idx_c.at[j]], rb, sem_b)
        da.wait()
        db.wait()
        pltpu.sync_copy(ra, ga_hbm.at[pl.ds(base + j * JW, JW)])
        pltpu.sync_copy(rb, gb_hbm.at[pl.ds(base + j * JW, JW)])
        return carry

    lax.fori_loop(0, J, body, 0)


_BN = 1000
_BE = 4000


def _tc1_body(x_ref, w_ref, degp_ref, y_ref, zs_ref, dis_ref):
    Y = jnp.dot(x_ref[...], w_ref[...], preferred_element_type=jnp.float32)
    deg = degp_ref[0, :, 0:1] + degp_ref[1, :, 0:1]
    dis = jnp.where(deg > 0, lax.rsqrt(jnp.where(deg > 0, deg, 1.0)), 0.0)
    for k in range(4):
        y_ref[k] = Y[:, k * H:(k + 1) * H]
    zs_ref[...] = dis * Y[:, 3 * H:4 * H]
    dis_ref[...] = dis


_tc1 = pl.pallas_call(
    _tc1_body,
    grid=(N // _BN,),
    in_specs=[
        pl.BlockSpec((_BN, D), lambda i: (i, 0)),
        pl.BlockSpec((D, 4 * H), lambda i: (0, 0)),
        pl.BlockSpec((NC, _BN, H), lambda i: (0, i, 0)),
    ],
    out_specs=[
        pl.BlockSpec((4, _BN, H), lambda i: (0, i, 0)),
        pl.BlockSpec((_BN, H), lambda i: (i, 0)),
        pl.BlockSpec((_BN, 1), lambda i: (i, 0)),
    ],
    out_shape=[
        jax.ShapeDtypeStruct((4, N, H), jnp.float32),
        jax.ShapeDtypeStruct((N, H), jnp.float32),
        jax.ShapeDtypeStruct((N, 1), jnp.float32),
    ],
)


def _comb_body(p_ref, y_ref, dis_ref, zs_ref):
    dis = dis_ref[...]
    z = y_ref[0] + dis * (p_ref[0] + p_ref[1])
    zs_ref[...] = dis * z


def _make_comb(k):
    return pl.pallas_call(
        _comb_body,
        grid=(N // _BN,),
        in_specs=[
            pl.BlockSpec((NC, _BN, H), lambda i: (0, i, 0)),
            pl.BlockSpec((1, _BN, H), lambda i, k=k: (k, i, 0)),
            pl.BlockSpec((_BN, 1), lambda i: (i, 0)),
        ],
        out_specs=pl.BlockSpec((_BN, H), lambda i: (i, 0)),
        out_shape=jax.ShapeDtypeStruct((N, H), jnp.float32),
    )


_comb_k2 = _make_comb(2)
_comb_k1 = _make_comb(1)


def _fin1_body(p_ref, y_ref, dis_ref, b_ref, w2_ref, y2_ref, zs_ref):
    dis = dis_ref[...]
    h1 = jnp.maximum(y_ref[0] + dis * (p_ref[0] + p_ref[1]) + b_ref[0:1, :], 0.0)
    Y2 = jnp.dot(h1, w2_ref[...], preferred_element_type=jnp.float32)
    for k in range(4):
        y2_ref[k] = Y2[:, k * H:(k + 1) * H]
    zs_ref[...] = dis * Y2[:, 3 * H:4 * H]


_fin1 = pl.pallas_call(
    _fin1_body,
    grid=(N // _BN,),
    in_specs=[
        pl.BlockSpec((NC, _BN, H), lambda i: (0, i, 0)),
        pl.BlockSpec((1, _BN, H), lambda i: (0, i, 0)),
        pl.BlockSpec((_BN, 1), lambda i: (i, 0)),
        pl.BlockSpec((8, H), lambda i: (0, 0)),
        pl.BlockSpec((H, 4 * H), lambda i: (0, 0)),
    ],
    out_specs=[
        pl.BlockSpec((4, _BN, H), lambda i: (0, i, 0)),
        pl.BlockSpec((_BN, H), lambda i: (i, 0)),
    ],
    out_shape=[
        jax.ShapeDtypeStruct((4, N, H), jnp.float32),
        jax.ShapeDtypeStruct((N, H), jnp.float32),
    ],
)


def _fin2_body(p_ref, y_ref, dis_ref, b_ref, wc_ref, bc_ref, a_ref, bb_ref):
    dis = dis_ref[...]
    h2 = jnp.maximum(y_ref[0] + dis * (p_ref[0] + p_ref[1]) + b_ref[0:1, :], 0.0)
    Wc = wc_ref[...]
    a_ref[...] = (jnp.dot(h2, Wc[:H], preferred_element_type=jnp.float32)
                  + bc_ref[0:1, :])
    bb_ref[...] = jnp.dot(h2, Wc[H:2 * H], preferred_element_type=jnp.float32)


_fin2 = pl.pallas_call(
    _fin2_body,
    grid=(N // _BN,),
    in_specs=[
        pl.BlockSpec((NC, _BN, H), lambda i: (0, i, 0)),
        pl.BlockSpec((1, _BN, H), lambda i: (0, i, 0)),
        pl.BlockSpec((_BN, 1), lambda i: (i, 0)),
        pl.BlockSpec((8, H), lambda i: (0, 0)),
        pl.BlockSpec((2 * H, C), lambda i: (0, 0)),
        pl.BlockSpec((8, C), lambda i: (0, 0)),
    ],
    out_specs=[
        pl.BlockSpec((_BN, C), lambda i: (i, 0)),
        pl.BlockSpec((_BN, C), lambda i: (i, 0)),
    ],
    out_shape=[
        jax.ShapeDtypeStruct((N, C), jnp.float32),
        jax.ShapeDtypeStruct((N, C), jnp.float32),
    ],
)


def _final_body(ga_ref, gb_ref, o_ref):
    t = ga_ref[...] + gb_ref[...]
    m = jnp.max(t, axis=1, keepdims=True)
    e = jnp.exp(t - m)
    o_ref[...] = (t - m) - jnp.log(jnp.sum(e, axis=1, keepdims=True))


_final = pl.pallas_call(
    _final_body,
    grid=(E // _BE,),
    in_specs=[
        pl.BlockSpec((_BE, C), lambda i: (i, 0)),
        pl.BlockSpec((_BE, C), lambda i: (i, 0)),
    ],
    out_specs=pl.BlockSpec((_BE, C), lambda i: (i, 0)),
    out_shape=jax.ShapeDtypeStruct((E, C), jnp.float32),
)


def kernel(x, edge_index, W1, b1, W2, b2, Wc, bc):
    ei = edge_index.astype(jnp.int32)
    row3 = ei[0].reshape(NW, J, JW)
    col3 = ei[1].reshape(NW, J, JW)
    W1c = jnp.transpose(W1, (1, 0, 2)).reshape(D, 4 * H)
    W2c = jnp.transpose(W2, (1, 0, 2)).reshape(H, 4 * H)
    b1t = jnp.tile(b1.reshape(1, H), (8, 1))
    b2t = jnp.tile(b2.reshape(1, H), (8, 1))
    bct = jnp.tile(bc.reshape(1, C), (8, 1))
    zeros_nh = jnp.zeros((NPAD, H), jnp.float32)
    ones_jw = jnp.ones((JW, H), jnp.float32)

    degp = _sc_deg(col3, zeros_nh, ones_jw)
    Y1, zs, dis = _tc1(x, W1c, degp)
    for comb in (_comb_k2, _comb_k1):
        p = _sc_hop(zs, row3, col3, zeros_nh)
        zs = comb(p, Y1, dis)
    p = _sc_hop(zs, row3, col3, zeros_nh)
    Y2, zs = _fin1(p, Y1, dis, b1t, W2c)
    for comb in (_comb_k2, _comb_k1):
        p = _sc_hop(zs, row3, col3, zeros_nh)
        zs = comb(p, Y2, dis)
    p = _sc_hop(zs, row3, col3, zeros_nh)
    a, bb = _fin2(p, Y2, dis, b2t, Wc, bct)
    ga, gb = _sc_edge(a, bb, row3, col3)
    return _final(ga, gb)
